# baseline (speedup 1.0000x reference)
import jax
import jax.numpy as jnp
from jax import lax
from jax.experimental import pallas as pl
from jax.experimental.pallas import tpu as pltpu
from jax.experimental.pallas import tpu_sc as plsc

DEPTH_N = 11
TREES = 8
N_NODES = 2 ** DEPTH_N - 1
ROWS = N_NODES * TREES
F = 1024
B = 8192
LANES = 16
GROUPS = F // (2 * LANES)
NC, NS = 2, 16
NW = NC * NS
TOK_W = B // NW
BLK_TOK = 2
ITERS = TOK_W // (2 * BLK_TOK)


def _forest_body(x_hbm, keys_hbm, values_hbm, y_hbm,
                 xbA, xbB, ybA, ybB, krA, krB, vrA, vrB,
                 idxA, idxB, accm, semA, semB):
    cid = lax.axis_index("c")
    sid = lax.axis_index("s")
    wid = sid * NC + cid
    tok0 = wid * TOK_W
    tree = lax.iota(jnp.int32, 16) & 7

    def issue(idx_ref, node, krows, vrows, sem):
        idx_ref[...] = node * TREES + tree
        ck = pltpu.async_copy(keys_hbm.at[idx_ref], krows, sem)
        cv = pltpu.async_copy(values_hbm.at[idx_ref], vrows, sem)
        return ck, cv

    def lam_of(krows, xb):
        def chunk(ci, accs):
            o = pl.multiple_of(ci * LANES, LANES)
            x0 = xb[0, pl.ds(o, LANES)]
            x1 = xb[1, pl.ds(o, LANES)]
            return tuple(
                accs[p] + krows[p, pl.ds(o, LANES)] * (x0 if p < 8 else x1)
                for p in range(16))
        accs = lax.fori_loop(
            0, 2 * GROUPS, chunk,
            tuple(jnp.zeros((LANES,), jnp.float32) for _ in range(16)))
        for p in range(16):
            accm[p, :] = accs[p]
        lane = lax.iota(jnp.int32, 16)
        lam = jnp.zeros((LANES,), jnp.float32)
        for j in range(16):
            lam = lam + plsc.load_gather(
                accm, [lane, jnp.full((16,), j, jnp.int32)])
        lam_s = [lam[k] for k in range(16)]
        return lam, lam_s

    def accum_y(yb, vrows, lam_s, first):
        def chunk(ci, c):
            o = pl.multiple_of(ci * LANES, LANES)
            if first:
                y0 = lam_s[0] * vrows[0, pl.ds(o, LANES)]
                y1 = lam_s[8] * vrows[8, pl.ds(o, LANES)]
                ks = range(1, 8)
            else:
                y0 = yb[0, pl.ds(o, LANES)]
                y1 = yb[1, pl.ds(o, LANES)]
                ks = range(8)
            for k in ks:
                y0 = y0 + lam_s[k] * vrows[k, pl.ds(o, LANES)]
                y1 = y1 + lam_s[k + 8] * vrows[k + 8, pl.ds(o, LANES)]
            yb[0, pl.ds(o, LANES)] = y0
            yb[1, pl.ds(o, LANES)] = y1
            return c
        lax.fori_loop(0, 2 * GROUPS, chunk, 0)

    def block(i, carry):
        baseA = tok0 + i * (2 * BLK_TOK)
        baseB = baseA + BLK_TOK
        pltpu.sync_copy(x_hbm.at[pl.ds(baseA, BLK_TOK)], xbA)
        pltpu.sync_copy(x_hbm.at[pl.ds(baseB, BLK_TOK)], xbB)
        nodeA = jnp.zeros((LANES,), jnp.int32)
        nodeB = jnp.zeros((LANES,), jnp.int32)
        dAk, dAv = issue(idxA, nodeA, krA, vrA, semA)
        for d in range(DEPTH_N):
            dBk, dBv = issue(idxB, nodeB, krB, vrB, semB)
            dAk.wait()
            dAv.wait()
            lamA, lamAs = lam_of(krA, xbA)
            nodeA = nodeA * 2 + 1 + (lamA > 0).astype(jnp.int32)
            accum_y(ybA, vrA, lamAs, first=(d == 0))
            if d < DEPTH_N - 1:
                dAk, dAv = issue(idxA, nodeA, krA, vrA, semA)
            dBk.wait()
            dBv.wait()
            lamB, lamBs = lam_of(krB, xbB)
            nodeB = nodeB * 2 + 1 + (lamB > 0).astype(jnp.int32)
            accum_y(ybB, vrB, lamBs, first=(d == 0))
        pltpu.sync_copy(ybA, y_hbm.at[pl.ds(baseA, BLK_TOK)])
        pltpu.sync_copy(ybB, y_hbm.at[pl.ds(baseB, BLK_TOK)])
        return carry

    lax.fori_loop(0, ITERS, block, 0)


@jax.jit
def kernel(x, keys, values):
    x, keys = jax.lax.optimization_barrier(
        (x.astype(jnp.bfloat16), keys.astype(jnp.bfloat16)))
    x = x.astype(jnp.float32)
    keys2 = keys.astype(jnp.float32).reshape(ROWS, F)
    values2 = values.reshape(ROWS, F)
    mesh = plsc.VectorSubcoreMesh(core_axis_name="c", subcore_axis_name="s")
    fk = pl.kernel(
        _forest_body,
        out_type=jax.ShapeDtypeStruct((B, F), jnp.float32),
        mesh=mesh,
        compiler_params=pltpu.CompilerParams(needs_layout_passes=False),
        scratch_types=[
            pltpu.VMEM((BLK_TOK, F), jnp.float32),
            pltpu.VMEM((BLK_TOK, F), jnp.float32),
            pltpu.VMEM((BLK_TOK, F), jnp.float32),
            pltpu.VMEM((BLK_TOK, F), jnp.float32),
            pltpu.VMEM((LANES, F), jnp.float32),
            pltpu.VMEM((LANES, F), jnp.float32),
            pltpu.VMEM((LANES, F), jnp.float32),
            pltpu.VMEM((LANES, F), jnp.float32),
            pltpu.VMEM((LANES,), jnp.int32),
            pltpu.VMEM((LANES,), jnp.int32),
            pltpu.VMEM((LANES, LANES), jnp.float32),
            pltpu.SemaphoreType.DMA,
            pltpu.SemaphoreType.DMA,
        ],
    )
    return fk(x, keys2, values2)


# speedup vs baseline: 1.0250x; 1.0250x over previous
import jax
import jax.numpy as jnp
from jax import lax
from jax.experimental import pallas as pl
from jax.experimental.pallas import tpu as pltpu
from jax.experimental.pallas import tpu_sc as plsc

DEPTH_N = 11
TREES = 8
N_NODES = 2 ** DEPTH_N - 1
ROWS = N_NODES * TREES
F = 1024
B = 8192
LANES = 16
GROUPS = F // (2 * LANES)
NC, NS = 2, 16
NW = NC * NS
TOK_W = B // NW
BLK_TOK = 2
ITERS = TOK_W // (2 * BLK_TOK)


def _forest_body(x_hbm, keys_hbm, values_hbm, y_hbm,
                 xbA, xbB, ybA, ybB, krA, krB, vrA, vrB,
                 idxA, idxB, semA, semB):
    cid = lax.axis_index("c")
    sid = lax.axis_index("s")
    wid = sid * NC + cid
    tok0 = wid * TOK_W
    tree = lax.iota(jnp.int32, 16) & 7

    def issue(idx_ref, node, krows, vrows, sem):
        idx_ref[...] = node * TREES + tree
        ck = pltpu.async_copy(keys_hbm.at[idx_ref], krows, sem)
        cv = pltpu.async_copy(values_hbm.at[idx_ref], vrows, sem)
        return ck, cv

    def lam_of(krows, xb):
        def chunk(ci, accs):
            o = pl.multiple_of(ci * LANES, LANES)
            x0 = xb[0, pl.ds(o, LANES)]
            x1 = xb[1, pl.ds(o, LANES)]
            return tuple(
                accs[p] + krows[p, pl.ds(o, LANES)] * (x0 if p < 8 else x1)
                for p in range(16))
        accs = lax.fori_loop(
            0, 2 * GROUPS, chunk,
            tuple(jnp.zeros((LANES,), jnp.float32) for _ in range(16)))
        lam_s = [jnp.sum(accs[p]) for p in range(16)]
        lane = lax.iota(jnp.int32, 16)
        lam = jnp.zeros((LANES,), jnp.float32)
        for p in range(16):
            lam = jnp.where(lane == p, lam_s[p], lam)
        return lam, lam_s

    def accum_y(yb, vrows, lam_s, first):
        def chunk(ci, c):
            o = pl.multiple_of(ci * LANES, LANES)
            if first:
                y0 = lam_s[0] * vrows[0, pl.ds(o, LANES)]
                y1 = lam_s[8] * vrows[8, pl.ds(o, LANES)]
                ks = range(1, 8)
            else:
                y0 = yb[0, pl.ds(o, LANES)]
                y1 = yb[1, pl.ds(o, LANES)]
                ks = range(8)
            for k in ks:
                y0 = y0 + lam_s[k] * vrows[k, pl.ds(o, LANES)]
                y1 = y1 + lam_s[k + 8] * vrows[k + 8, pl.ds(o, LANES)]
            yb[0, pl.ds(o, LANES)] = y0
            yb[1, pl.ds(o, LANES)] = y1
            return c
        lax.fori_loop(0, 2 * GROUPS, chunk, 0)

    def block(i, carry):
        baseA = tok0 + i * (2 * BLK_TOK)
        baseB = baseA + BLK_TOK
        pltpu.sync_copy(x_hbm.at[pl.ds(baseA, BLK_TOK)], xbA)
        pltpu.sync_copy(x_hbm.at[pl.ds(baseB, BLK_TOK)], xbB)
        nodeA = jnp.zeros((LANES,), jnp.int32)
        nodeB = jnp.zeros((LANES,), jnp.int32)
        dAk, dAv = issue(idxA, nodeA, krA, vrA, semA)
        for d in range(DEPTH_N):
            dBk, dBv = issue(idxB, nodeB, krB, vrB, semB)
            dAk.wait()
            dAv.wait()
            lamA, lamAs = lam_of(krA, xbA)
            nodeA = nodeA * 2 + 1 + (lamA > 0).astype(jnp.int32)
            accum_y(ybA, vrA, lamAs, first=(d == 0))
            if d < DEPTH_N - 1:
                dAk, dAv = issue(idxA, nodeA, krA, vrA, semA)
            dBk.wait()
            dBv.wait()
            lamB, lamBs = lam_of(krB, xbB)
            nodeB = nodeB * 2 + 1 + (lamB > 0).astype(jnp.int32)
            accum_y(ybB, vrB, lamBs, first=(d == 0))
        pltpu.sync_copy(ybA, y_hbm.at[pl.ds(baseA, BLK_TOK)])
        pltpu.sync_copy(ybB, y_hbm.at[pl.ds(baseB, BLK_TOK)])
        return carry

    lax.fori_loop(0, ITERS, block, 0)


@jax.jit
def kernel(x, keys, values):
    x, keys = jax.lax.optimization_barrier(
        (x.astype(jnp.bfloat16), keys.astype(jnp.bfloat16)))
    x = x.astype(jnp.float32)
    keys2 = keys.astype(jnp.float32).reshape(ROWS, F)
    values2 = values.reshape(ROWS, F)
    mesh = plsc.VectorSubcoreMesh(core_axis_name="c", subcore_axis_name="s")
    fk = pl.kernel(
        _forest_body,
        out_type=jax.ShapeDtypeStruct((B, F), jnp.float32),
        mesh=mesh,
        compiler_params=pltpu.CompilerParams(needs_layout_passes=False),
        scratch_types=[
            pltpu.VMEM((BLK_TOK, F), jnp.float32),
            pltpu.VMEM((BLK_TOK, F), jnp.float32),
            pltpu.VMEM((BLK_TOK, F), jnp.float32),
            pltpu.VMEM((BLK_TOK, F), jnp.float32),
            pltpu.VMEM((LANES, F), jnp.float32),
            pltpu.VMEM((LANES, F), jnp.float32),
            pltpu.VMEM((LANES, F), jnp.float32),
            pltpu.VMEM((LANES, F), jnp.float32),
            pltpu.VMEM((LANES,), jnp.int32),
            pltpu.VMEM((LANES,), jnp.int32),
            pltpu.SemaphoreType.DMA,
            pltpu.SemaphoreType.DMA,
        ],
    )
    return fk(x, keys2, values2)
